# Initial kernel scaffold; baseline (speedup 1.0000x reference)
import jax
import jax.numpy as jnp
from jax.experimental import pallas as pl


def kernel(seq, embed, W1, b1, W2, b2, gamma, beta, Wq, bq, Wout, bout):
    raise NotImplementedError("write your pallas kernel here")


# speedup vs baseline: 120.5151x; 120.5151x over previous
import jax
import jax.numpy as jnp
from jax import lax
from jax.experimental import pallas as pl
from jax.experimental.pallas import tpu as pltpu

_H = 64
_V = 64
_S = 64
_HI = jax.lax.Precision.HIGHEST
_CHUNK = 512


def _body(seq_ref, embed_ref, W1_ref, b1_ref, W2_ref, b2_ref, gamma_ref,
          beta_ref, Wq_ref, bq_ref, Wout_ref, bout_ref, out_ref, acc_ref):
    i = pl.program_id(0)
    nsteps = pl.num_programs(0)
    B, C = seq_ref.shape
    L = C * nsteps

    @pl.when(i == 0)
    def _init():
        acc_ref[...] = jnp.zeros_like(acc_ref)

    x = seq_ref[...]
    pos = i * C + lax.broadcasted_iota(jnp.int32, (B, C), 1)
    valid = pos < (L - 3)
    for v in range(_V):
        hit = jnp.where((x == v) & valid, 1.0, 0.0)
        acc_ref[:, v:v + 1] += jnp.sum(hit, axis=1, keepdims=True)

    @pl.when(i == nsteps - 1)
    def _finish():
        counts = acc_ref[...]

        embed = embed_ref[...]
        ff = jnp.maximum(
            jax.lax.dot_general(embed, W1_ref[...], (((1,), (0,)), ((), ())),
                                precision=_HI) + b1_ref[...], 0.0)
        ff = jax.lax.dot_general(ff, W2_ref[...], (((1,), (0,)), ((), ())),
                                 precision=_HI) + b2_ref[...]
        h = embed + ff
        mu = jnp.mean(h, axis=-1, keepdims=True)
        var = jnp.mean((h - mu) ** 2, axis=-1, keepdims=True)
        T = gamma_ref[...] * (h - mu) / jnp.sqrt(var + 1e-5) + beta_ref[...]

        nsq = jnp.sum(T * T, axis=-1)
        gt = (nsq[:, None] > nsq[None, :]).astype(jnp.float32)
        prior = jax.lax.dot_general(counts, gt, (((1,), (0,)), ((), ())),
                                    precision=_HI)
        prior = jnp.round(prior)
        c = jnp.clip(float(_S) - prior, 0.0, counts)

        t_last = x[:, C - 1][:, None]
        onehot = (t_last == lax.broadcasted_iota(jnp.int32, (B, _V), 1)
                  ).astype(jnp.float32)
        hs_last = jax.lax.dot_general(onehot, T, (((1,), (0,)), ((), ())),
                                      precision=_HI)
        q = jax.lax.dot_general(hs_last, Wq_ref[...],
                                (((1,), (0,)), ((), ())),
                                precision=_HI) + bq_ref[...]

        logits = jax.lax.dot_general(q, T, (((1,), (1,)), ((), ())),
                                     precision=_HI) * (1.0 / (_H ** 0.5))
        m = jnp.max(jnp.where(c > 0, logits, -1e30), axis=1, keepdims=True)
        w = c * jnp.exp(logits - m)
        ctx = jax.lax.dot_general(w, T, (((1,), (0,)), ((), ())),
                                  precision=_HI) / jnp.sum(w, axis=1,
                                                           keepdims=True)
        out_ref[...] = jax.lax.dot_general(
            ctx, Wout_ref[...], (((1,), (0,)), ((), ())),
            precision=_HI) + bout_ref[...]


def kernel(seq, embed, W1, b1, W2, b2, gamma, beta, Wq, bq, Wout, bout):
    B, L = seq.shape
    nsteps = L // _CHUNK
    full = lambda i: (0, 0)
    vec = lambda i: (0,)
    out = pl.pallas_call(
        _body,
        grid=(nsteps,),
        in_specs=[
            pl.BlockSpec((B, _CHUNK), lambda i: (0, i)),
            pl.BlockSpec(embed.shape, full),
            pl.BlockSpec(W1.shape, full),
            pl.BlockSpec(b1.shape, vec),
            pl.BlockSpec(W2.shape, full),
            pl.BlockSpec(b2.shape, vec),
            pl.BlockSpec(gamma.shape, vec),
            pl.BlockSpec(beta.shape, vec),
            pl.BlockSpec(Wq.shape, full),
            pl.BlockSpec(bq.shape, vec),
            pl.BlockSpec(Wout.shape, full),
            pl.BlockSpec(bout.shape, vec),
        ],
        out_specs=pl.BlockSpec((B, _V), full),
        out_shape=jax.ShapeDtypeStruct((B, _V), jnp.float32),
        scratch_shapes=[pltpu.VMEM((B, _V), jnp.float32)],
    )(seq, embed, W1, b1, W2, b2, gamma, beta, Wq, bq, Wout, bout)
    return out


# speedup vs baseline: 122.4390x; 1.0160x over previous
import functools

import jax
import jax.numpy as jnp
from jax import lax
from jax.experimental import pallas as pl
from jax.experimental.pallas import tpu as pltpu
from jax.experimental.pallas import tpu_sc as plsc

_H = 64
_V = 64
_S = 64
_HI = jax.lax.Precision.HIGHEST

_B, _LSEQ = 128, 8192
_NC, _NS, _LANES = 2, 16, 16
_NW = _NC * _NS
_RPW = _B // _NW
_NH = _LSEQ - 3

@functools.lru_cache(maxsize=None)
def _make_hist_sc():
    mesh = plsc.VectorSubcoreMesh(core_axis_name="c", subcore_axis_name="s",
                                  num_cores=_NC, num_subcores=_NS)
    return functools.partial(
        pl.kernel,
        out_type=jax.ShapeDtypeStruct((_B, _V), jnp.float32),
        mesh=mesh,
        scratch_types=[
            pltpu.VMEM((_RPW, _LSEQ), jnp.int32),
            pltpu.VMEM((_RPW * _LANES * _V,), jnp.float32),
            pltpu.VMEM((_RPW, _V), jnp.float32),
        ],
        compiler_params=pltpu.CompilerParams(needs_layout_passes=False),
    )(_hist_sc)


def _hist_sc(seq_hbm, out_hbm, toks_v, hist_v, cnt_v):
    wid = lax.axis_index("s") * _NC + lax.axis_index("c")
    base = wid * _RPW
    pltpu.sync_copy(seq_hbm.at[pl.ds(base, _RPW)], toks_v)

    lane = lax.broadcasted_iota(jnp.int32, (_LANES,), 0)
    ones = jnp.ones((_LANES,), jnp.float32)
    zeros = jnp.zeros((_LANES,), jnp.float32)
    lane_base = lane * _V
    nfull = _NH // _LANES
    rem = _NH - nfull * _LANES

    for j in range(_RPW * _V):
        hist_v[pl.ds(j * _LANES, _LANES)] = zeros

    for r in range(_RPW):
        roff = r * _LANES * _V

        def chunk(i, _, r=r, roff=roff):
            x = toks_v[r, pl.ds(i * _LANES, _LANES)]
            plsc.addupdate_scatter(hist_v, [roff + lane_base + x], ones)
            return 0

        lax.fori_loop(0, nfull, chunk, 0, unroll=8)
        x = toks_v[r, pl.ds(nfull * _LANES, _LANES)]
        plsc.addupdate_scatter(hist_v, [roff + lane_base + x], ones,
                               mask=lane < rem)

        for j in range(_V // _LANES):
            acc = zeros
            for l in range(_LANES):
                acc = acc + hist_v[pl.ds(roff + l * _V + j * _LANES, _LANES)]
            cnt_v[r, pl.ds(j * _LANES, _LANES)] = acc

    pltpu.sync_copy(cnt_v, out_hbm.at[pl.ds(base, _RPW)])


def _attn_body(counts_ref, tlast_ref, embed_ref, W1_ref, b1_ref, W2_ref,
               b2_ref, gamma_ref, beta_ref, Wq_ref, bq_ref, Wout_ref,
               bout_ref, out_ref):
    counts = counts_ref[...]
    B = counts.shape[0]

    embed = embed_ref[...]
    ff = jnp.maximum(
        jax.lax.dot_general(embed, W1_ref[...], (((1,), (0,)), ((), ())),
                            precision=_HI) + b1_ref[...], 0.0)
    ff = jax.lax.dot_general(ff, W2_ref[...], (((1,), (0,)), ((), ())),
                             precision=_HI) + b2_ref[...]
    h = embed + ff
    mu = jnp.mean(h, axis=-1, keepdims=True)
    var = jnp.mean((h - mu) ** 2, axis=-1, keepdims=True)
    T = gamma_ref[...] * (h - mu) / jnp.sqrt(var + 1e-5) + beta_ref[...]

    nsq = jnp.sum(T * T, axis=-1)
    gt = (nsq[:, None] > nsq[None, :]).astype(jnp.float32)
    prior = jax.lax.dot_general(counts, gt, (((1,), (0,)), ((), ())),
                                precision=_HI)
    prior = jnp.round(prior)
    c = jnp.clip(float(_S) - prior, 0.0, counts)

    t_last = tlast_ref[...]
    onehot = (t_last == lax.broadcasted_iota(jnp.int32, (B, _V), 1)
              ).astype(jnp.float32)
    hs_last = jax.lax.dot_general(onehot, T, (((1,), (0,)), ((), ())),
                                  precision=_HI)
    q = jax.lax.dot_general(hs_last, Wq_ref[...], (((1,), (0,)), ((), ())),
                            precision=_HI) + bq_ref[...]

    logits = jax.lax.dot_general(q, T, (((1,), (1,)), ((), ())),
                                 precision=_HI) * (1.0 / (_H ** 0.5))
    m = jnp.max(jnp.where(c > 0, logits, -1e30), axis=1, keepdims=True)
    w = c * jnp.exp(logits - m)
    ctx = jax.lax.dot_general(w, T, (((1,), (0,)), ((), ())),
                              precision=_HI) / jnp.sum(w, axis=1,
                                                       keepdims=True)
    out_ref[...] = jax.lax.dot_general(
        ctx, Wout_ref[...], (((1,), (0,)), ((), ())),
        precision=_HI) + bout_ref[...]


def kernel(seq, embed, W1, b1, W2, b2, gamma, beta, Wq, bq, Wout, bout):
    counts = _make_hist_sc()(seq)
    t_last = seq[:, -1:]
    out = pl.pallas_call(
        _attn_body,
        out_shape=jax.ShapeDtypeStruct((_B, _V), jnp.float32),
    )(counts, t_last, embed, W1, b1, W2, b2, gamma, beta, Wq, bq,
      Wout, bout)
    return out
